# baseline (speedup 1.0000x reference)
import functools

import jax
import jax.numpy as jnp
import numpy as np
from jax import lax
from jax.experimental import pallas as pl
from jax.experimental.pallas import tpu as pltpu
from jax.experimental.pallas import tpu_sc as plsc

N = 10000
E = 160000
C = 128
_INV_SQRT3 = float(1.0 / np.sqrt(3.0))
_INV_SQRT2 = float(1.0 / np.sqrt(2.0))

_NTILE = 16
_EDGES_PER_TILE = E // _NTILE
_EB = 80
_NBLK = _EDGES_PER_TILE // _EB
_NPAD = 10240
_ROWS_PER_TILE = _NPAD // _NTILE
_ZROWS = 40


def _sc_message_passing(xflat, edge_vals, src, dst):
    mesh = plsc.VectorSubcoreMesh(core_axis_name="c", subcore_axis_name="s")

    vm = pltpu.VMEM
    @functools.partial(
        pl.kernel,
        out_type=jax.ShapeDtypeStruct((4 * _NPAD, C), jnp.float32),
        mesh=mesh,
        scratch_types=[
            vm((_EB,), jnp.int32), vm((_EB,), jnp.int32),
            vm((_EB,), jnp.int32), vm((_EB,), jnp.int32),
            vm((_EB,), jnp.int32), vm((_EB,), jnp.int32),
            vm((_EB,), jnp.int32), vm((_EB,), jnp.int32),
            vm((_EB, C), jnp.float32), vm((_EB, C), jnp.float32),
            vm((_EB, C), jnp.float32), vm((_EB, C), jnp.float32),
            vm((_ZROWS, C), jnp.float32),
            pltpu.VMEM_SHARED((_NPAD, C), jnp.float32),
            pltpu.SemaphoreType.DMA, pltpu.SemaphoreType.DMA,
            pltpu.SemaphoreType.DMA, pltpu.SemaphoreType.DMA,
            pltpu.SemaphoreType.DMA, pltpu.SemaphoreType.DMA,
            pltpu.SemaphoreType.DMA, pltpu.SemaphoreType.DMA,
            pltpu.SemaphoreType.DMA, pltpu.SemaphoreType.DMA,
        ],
    )
    def body(x_hbm, vals_hbm, src_hbm, dst_hbm, y_hbm,
             srcv0, srcv1, dstv0, dstv1, srcov0, srcov1, dsts0, dsts1,
             rowv0, rowv1, valv0, valv1, zerov, acc,
             ssrc0, ssrc1, sdst0, sdst1, sval0, sval1, sgat0, sgat1,
             ssct0, ssct1):
        cid = lax.axis_index("c")
        sid = lax.axis_index("s")
        bufs = [(srcv0, dstv0, srcov0, dsts0, rowv0, valv0,
                 ssrc0, sdst0, sval0, sgat0, ssct0),
                (srcv1, dstv1, srcov1, dsts1, rowv1, valv1,
                 ssrc1, sdst1, sval1, sgat1, ssct1)]

        def zfill(r, carry):
            for k in range(C // 16):
                zerov[r, pl.ds(k * 16, 16)] = jnp.zeros((16,), jnp.float32)
            return carry
        lax.fori_loop(0, _ZROWS, zfill, 0)

        def e0_of(b):
            return sid * _EDGES_PER_TILE + b * _EB

        def fire_idx(b, p):
            srcv, dstv, _, _, _, valv, ssrc, sdst, sval, _, _ = bufs[p]
            e0 = e0_of(b)
            pltpu.async_copy(src_hbm.at[pl.ds(e0, _EB)], srcv, ssrc)
            pltpu.async_copy(dst_hbm.at[pl.ds(e0, _EB)], dstv, sdst)
            pltpu.async_copy(vals_hbm.at[pl.ds(e0, _EB), :], valv, sval)

        def drain_scatter(p):
            _, _, _, dsts, rowv, _, _, _, _, _, ssct = bufs[p]
            pltpu.make_async_copy(rowv, acc.at[dsts], ssct).wait()

        def wait_idx_fire_gather(b, p, slot_off, drain):
            srcv, _, srcov, _, rowv, _, ssrc, _, _, sgat, _ = bufs[p]
            e0 = e0_of(b)
            pltpu.make_async_copy(src_hbm.at[pl.ds(e0, _EB)],
                                  srcv, ssrc).wait()
            for k in range(_EB // 16):
                srcov[pl.ds(k * 16, 16)] = srcv[pl.ds(k * 16, 16)] + slot_off
            if drain:
                drain_scatter(p)
            pltpu.async_copy(x_hbm.at[srcov], rowv, sgat)

        def consume(b, p):
            _, dstv, srcov, dsts, rowv, valv, _, sdst, sval, sgat, ssct = \
                bufs[p]
            e0 = e0_of(b)
            pltpu.make_async_copy(dst_hbm.at[pl.ds(e0, _EB)],
                                  dstv, sdst).wait()
            pltpu.make_async_copy(vals_hbm.at[pl.ds(e0, _EB), :],
                                  valv, sval).wait()
            pltpu.make_async_copy(x_hbm.at[srcov], rowv, sgat).wait()
            for k in range(_EB // 16):
                dsts[pl.ds(k * 16, 16)] = dstv[pl.ds(k * 16, 16)]

            @plsc.parallel_loop(0, _EB, unroll=8)
            def rmul(r):
                for k in range(C // 16):
                    rowv[r, pl.ds(k * 16, 16)] = (
                        rowv[r, pl.ds(k * 16, 16)] * valv[r, pl.ds(k * 16, 16)])
            pltpu.async_copy(rowv, acc.at[dsts], ssct, add=True)

        for j in range(2):
            slot = cid * 2 + j
            slot_off = slot * N
            for z in range(_ROWS_PER_TILE // _ZROWS):
                r0 = sid * _ROWS_PER_TILE + z * _ZROWS
                pltpu.sync_copy(zerov, acc.at[pl.ds(r0, _ZROWS), :])
            plsc.subcore_barrier()

            fire_idx(0, 0)
            wait_idx_fire_gather(0, 0, slot_off, False)
            fire_idx(1, 1)
            wait_idx_fire_gather(1, 1, slot_off, False)
            consume(0, 0)
            fire_idx(2, 0)

            def pair(t, carry):
                for q in range(2):
                    b = 2 + 2 * t + q
                    p = q
                    wait_idx_fire_gather(b, p, slot_off, True)
                    consume(b - 1, 1 - p)
                    fire_idx(b + 1, 1 - p)
                return carry
            lax.fori_loop(0, (_NBLK - 3) // 2, pair, 0)

            b_last = _NBLK - 1
            wait_idx_fire_gather(b_last, b_last % 2, slot_off, True)
            consume(b_last - 1, 1 - b_last % 2)
            consume(b_last, b_last % 2)
            drain_scatter(1 - b_last % 2)
            drain_scatter(b_last % 2)
            plsc.subcore_barrier()

            r0 = sid * _ROWS_PER_TILE
            pltpu.sync_copy(acc.at[pl.ds(r0, _ROWS_PER_TILE), :],
                            y_hbm.at[pl.ds(slot * _NPAD + r0, _ROWS_PER_TILE), :])
            plsc.subcore_barrier()

    return body(xflat, edge_vals, src, dst)


_BN = 400


def _tc_dense(y4, x4, w0, w1):
    def body(y_ref, x_ref, w0_ref, w1_ref, o0_ref, o1_ref):
        y0 = y_ref[0]
        ya, yb, yc = y_ref[1], y_ref[2], y_ref[3]
        x0 = x_ref[0]
        xa, xb, xc = x_ref[1], x_ref[2], x_ref[3]

        f0 = jnp.concatenate([
            y0 * y0, (ya * ya + yb * yb + yc * yc) * _INV_SQRT3,
            y0 * x0, (ya * xa + yb * xb + yc * xc) * _INV_SQRT3,
            x0 * x0, (xa * xa + xb * xb + xc * xc) * _INV_SQRT3,
        ], axis=1)
        o0_ref[...] = jnp.dot(f0, w0_ref[...],
                              preferred_element_type=jnp.float32) + x0

        cra = (yb * xc - yc * xb) * _INV_SQRT2
        crb = (yc * xa - ya * xc) * _INV_SQRT2
        crc = (ya * xb - yb * xa) * _INV_SQRT2
        f1a = jnp.concatenate([y0 * ya, y0 * xa, ya * x0, cra, x0 * xa], axis=1)
        f1b = jnp.concatenate([y0 * yb, y0 * xb, yb * x0, crb, x0 * xb], axis=1)
        f1c = jnp.concatenate([y0 * yc, y0 * xc, yc * x0, crc, x0 * xc], axis=1)
        f1 = jnp.concatenate([f1a, f1b, f1c], axis=0)
        g1 = jnp.dot(f1, w1_ref[...], preferred_element_type=jnp.float32)
        o1_ref[...] = g1.reshape(3, _BN, C) + jnp.stack([xa, xb, xc], axis=0)

    return pl.pallas_call(
        body,
        grid=(N // _BN,),
        in_specs=[
            pl.BlockSpec((4, _BN, C), lambda i: (0, i, 0)),
            pl.BlockSpec((4, _BN, C), lambda i: (0, i, 0)),
            pl.BlockSpec((6 * C, C), lambda i: (0, 0)),
            pl.BlockSpec((5 * C, C), lambda i: (0, 0)),
        ],
        out_specs=[
            pl.BlockSpec((_BN, C), lambda i: (i, 0)),
            pl.BlockSpec((3, _BN, C), lambda i: (0, i, 0)),
        ],
        out_shape=[
            jax.ShapeDtypeStruct((N, C), jnp.float32),
            jax.ShapeDtypeStruct((3, N, C), jnp.float32),
        ],
    )(y4, x4, w0, w1)


def kernel(x_l0, x_l1, edge_vals, edge_idx, W_mpmp_0, W_mpmp_1,
           W_mpid_0, W_mpid_1, W_idid_0, W_idid_1):
    src = edge_idx[0]
    dst = edge_idx[1]
    x4 = jnp.stack([x_l0[:, :, 0], x_l1[:, :, 0], x_l1[:, :, 1], x_l1[:, :, 2]],
                   axis=0)
    yflat = _sc_message_passing(x4.reshape(4 * N, C), edge_vals, src, dst)
    y4 = yflat.reshape(4, _NPAD, C)[:, :N]

    w0 = jnp.concatenate([W_mpmp_0, W_mpid_0, W_idid_0], axis=0)
    w_yy1 = W_mpmp_1[:C] + W_mpmp_1[C:2 * C]
    w_xx1 = W_idid_1[:C] + W_idid_1[C:2 * C]
    w1 = jnp.concatenate([w_yy1, W_mpid_1, w_xx1], axis=0)

    o0, o1 = _tc_dense(y4, x4, w0, w1)
    out0 = o0[:, :, None]
    out1 = jnp.transpose(o1, (1, 2, 0))
    return out0, out1


# speedup vs baseline: 1.0717x; 1.0717x over previous
import functools

import jax
import jax.numpy as jnp
import numpy as np
from jax import lax
from jax.experimental import pallas as pl
from jax.experimental.pallas import tpu as pltpu
from jax.experimental.pallas import tpu_sc as plsc

N = 10000
E = 160000
C = 128
_INV_SQRT3 = float(1.0 / np.sqrt(3.0))
_INV_SQRT2 = float(1.0 / np.sqrt(2.0))

_NTILE = 16
_EDGES_PER_TILE = E // _NTILE
_EB = 80
_NBLK = _EDGES_PER_TILE // _EB
_NPAD = 10240
_ROWS_PER_TILE = _NPAD // _NTILE
_ZROWS = 40


def _sc_message_passing(xflat, edge_vals, src, dst):
    mesh = plsc.VectorSubcoreMesh(core_axis_name="c", subcore_axis_name="s")

    vm = pltpu.VMEM
    @functools.partial(
        pl.kernel,
        out_type=jax.ShapeDtypeStruct((4 * _NPAD, C), jnp.float32),
        mesh=mesh,
        scratch_types=[
            vm((_EB,), jnp.int32), vm((_EB,), jnp.int32),
            vm((_EB,), jnp.int32), vm((_EB,), jnp.int32),
            vm((_EB,), jnp.int32), vm((_EB,), jnp.int32),
            vm((_EB,), jnp.int32), vm((_EB,), jnp.int32),
            vm((_EB, C), jnp.float32), vm((_EB, C), jnp.float32),
            vm((_EB, C), jnp.float32), vm((_EB, C), jnp.float32),
            vm((_ZROWS, C), jnp.float32),
            pltpu.VMEM_SHARED((_NPAD, C), jnp.float32),
            pltpu.SemaphoreType.DMA, pltpu.SemaphoreType.DMA,
            pltpu.SemaphoreType.DMA, pltpu.SemaphoreType.DMA,
            pltpu.SemaphoreType.DMA, pltpu.SemaphoreType.DMA,
            pltpu.SemaphoreType.DMA, pltpu.SemaphoreType.DMA,
            pltpu.SemaphoreType.DMA, pltpu.SemaphoreType.DMA,
        ],
    )
    def body(x_hbm, vals_hbm, src_hbm, dst_hbm, y_hbm,
             srcv0, srcv1, dstv0, dstv1, srcov0, srcov1, dsts0, dsts1,
             rowv0, rowv1, valv0, valv1, zerov, acc,
             ssrc0, ssrc1, sdst0, sdst1, sval0, sval1, sgat0, sgat1,
             ssct0, ssct1):
        cid = lax.axis_index("c")
        sid = lax.axis_index("s")
        bufs = [(srcv0, dstv0, srcov0, dsts0, rowv0, valv0,
                 ssrc0, sdst0, sval0, sgat0, ssct0),
                (srcv1, dstv1, srcov1, dsts1, rowv1, valv1,
                 ssrc1, sdst1, sval1, sgat1, ssct1)]

        def zfill(r, carry):
            for k in range(C // 16):
                zerov[r, pl.ds(k * 16, 16)] = jnp.zeros((16,), jnp.float32)
            return carry
        lax.fori_loop(0, _ZROWS, zfill, 0)

        def e0_of(b):
            return sid * _EDGES_PER_TILE + b * _EB

        def fire_idx(b, p):
            srcv, dstv, _, _, _, valv, ssrc, sdst, sval, _, _ = bufs[p]
            e0 = e0_of(b)
            pltpu.async_copy(src_hbm.at[pl.ds(e0, _EB)], srcv, ssrc)
            pltpu.async_copy(dst_hbm.at[pl.ds(e0, _EB)], dstv, sdst)
            pltpu.async_copy(vals_hbm.at[pl.ds(e0, _EB), :], valv, sval)

        def drain_scatter(p):
            _, _, _, dsts, rowv, _, _, _, _, _, ssct = bufs[p]
            pltpu.make_async_copy(rowv, acc.at[dsts], ssct).wait()

        def wait_idx_fire_gather(b, p, slot_off, drain):
            srcv, _, srcov, _, rowv, _, ssrc, _, _, sgat, _ = bufs[p]
            e0 = e0_of(b)
            pltpu.make_async_copy(src_hbm.at[pl.ds(e0, _EB)],
                                  srcv, ssrc).wait()
            for k in range(_EB // 16):
                srcov[pl.ds(k * 16, 16)] = srcv[pl.ds(k * 16, 16)] + slot_off
            if drain:
                drain_scatter(p)
            pltpu.async_copy(x_hbm.at[srcov], rowv, sgat)

        def consume(b, p):
            _, dstv, srcov, dsts, rowv, valv, _, sdst, sval, sgat, ssct = \
                bufs[p]
            e0 = e0_of(b)
            pltpu.make_async_copy(dst_hbm.at[pl.ds(e0, _EB)],
                                  dstv, sdst).wait()
            pltpu.make_async_copy(vals_hbm.at[pl.ds(e0, _EB), :],
                                  valv, sval).wait()
            pltpu.make_async_copy(x_hbm.at[srcov], rowv, sgat).wait()
            for k in range(_EB // 16):
                dsts[pl.ds(k * 16, 16)] = dstv[pl.ds(k * 16, 16)]

            @plsc.parallel_loop(0, _EB, unroll=4)
            def rmul(r):
                for k in range(C // 16):
                    rowv[r, pl.ds(k * 16, 16)] = (
                        rowv[r, pl.ds(k * 16, 16)] * valv[r, pl.ds(k * 16, 16)])
            pltpu.async_copy(rowv, acc.at[dsts], ssct, add=True)

        for j in range(2):
            slot = cid * 2 + j
            slot_off = slot * N
            for z in range(_ROWS_PER_TILE // _ZROWS):
                r0 = sid * _ROWS_PER_TILE + z * _ZROWS
                pltpu.sync_copy(zerov, acc.at[pl.ds(r0, _ZROWS), :])
            plsc.subcore_barrier()

            fire_idx(0, 0)
            wait_idx_fire_gather(0, 0, slot_off, False)
            fire_idx(1, 1)
            wait_idx_fire_gather(1, 1, slot_off, False)
            consume(0, 0)
            fire_idx(2, 0)

            def pair(t, carry):
                for q in range(2):
                    b = 2 + 2 * t + q
                    p = q
                    wait_idx_fire_gather(b, p, slot_off, True)
                    consume(b - 1, 1 - p)
                    fire_idx(b + 1, 1 - p)
                return carry
            lax.fori_loop(0, (_NBLK - 3) // 2, pair, 0)

            b_last = _NBLK - 1
            wait_idx_fire_gather(b_last, b_last % 2, slot_off, True)
            consume(b_last - 1, 1 - b_last % 2)
            consume(b_last, b_last % 2)
            drain_scatter(1 - b_last % 2)
            drain_scatter(b_last % 2)
            plsc.subcore_barrier()

            r0 = sid * _ROWS_PER_TILE
            pltpu.sync_copy(acc.at[pl.ds(r0, _ROWS_PER_TILE), :],
                            y_hbm.at[pl.ds(slot * _NPAD + r0, _ROWS_PER_TILE), :])
            plsc.subcore_barrier()

    return body(xflat, edge_vals, src, dst)


_BN = 1000


def _tc_dense(y4, x4, w0, w1):
    def body(y_ref, x_ref, w0_ref, w1_ref, o0_ref, o1_ref):
        y0 = y_ref[0]
        ya, yb, yc = y_ref[1], y_ref[2], y_ref[3]
        x0 = x_ref[0]
        xa, xb, xc = x_ref[1], x_ref[2], x_ref[3]

        f0 = jnp.concatenate([
            y0 * y0, (ya * ya + yb * yb + yc * yc) * _INV_SQRT3,
            y0 * x0, (ya * xa + yb * xb + yc * xc) * _INV_SQRT3,
            x0 * x0, (xa * xa + xb * xb + xc * xc) * _INV_SQRT3,
        ], axis=1)
        o0_ref[...] = jnp.dot(f0, w0_ref[...],
                              preferred_element_type=jnp.float32) + x0

        cra = (yb * xc - yc * xb) * _INV_SQRT2
        crb = (yc * xa - ya * xc) * _INV_SQRT2
        crc = (ya * xb - yb * xa) * _INV_SQRT2
        f1a = jnp.concatenate([y0 * ya, y0 * xa, ya * x0, cra, x0 * xa], axis=1)
        f1b = jnp.concatenate([y0 * yb, y0 * xb, yb * x0, crb, x0 * xb], axis=1)
        f1c = jnp.concatenate([y0 * yc, y0 * xc, yc * x0, crc, x0 * xc], axis=1)
        f1 = jnp.concatenate([f1a, f1b, f1c], axis=0)
        g1 = jnp.dot(f1, w1_ref[...], preferred_element_type=jnp.float32)
        o1_ref[...] = g1.reshape(3, _BN, C) + jnp.stack([xa, xb, xc], axis=0)

    return pl.pallas_call(
        body,
        grid=(N // _BN,),
        in_specs=[
            pl.BlockSpec((4, _BN, C), lambda i: (0, i, 0)),
            pl.BlockSpec((4, _BN, C), lambda i: (0, i, 0)),
            pl.BlockSpec((6 * C, C), lambda i: (0, 0)),
            pl.BlockSpec((5 * C, C), lambda i: (0, 0)),
        ],
        out_specs=[
            pl.BlockSpec((_BN, C), lambda i: (i, 0)),
            pl.BlockSpec((3, _BN, C), lambda i: (0, i, 0)),
        ],
        out_shape=[
            jax.ShapeDtypeStruct((N, C), jnp.float32),
            jax.ShapeDtypeStruct((3, N, C), jnp.float32),
        ],
    )(y4, x4, w0, w1)


def kernel(x_l0, x_l1, edge_vals, edge_idx, W_mpmp_0, W_mpmp_1,
           W_mpid_0, W_mpid_1, W_idid_0, W_idid_1):
    src = edge_idx[0]
    dst = edge_idx[1]
    x4 = jnp.stack([x_l0[:, :, 0], x_l1[:, :, 0], x_l1[:, :, 1], x_l1[:, :, 2]],
                   axis=0)
    yflat = _sc_message_passing(x4.reshape(4 * N, C), edge_vals, src, dst)
    y4 = yflat.reshape(4, _NPAD, C)[:, :N]

    w0 = jnp.concatenate([W_mpmp_0, W_mpid_0, W_idid_0], axis=0)
    w_yy1 = W_mpmp_1[:C] + W_mpmp_1[C:2 * C]
    w_xx1 = W_idid_1[:C] + W_idid_1[C:2 * C]
    w1 = jnp.concatenate([w_yy1, W_mpid_1, w_xx1], axis=0)

    o0, o1 = _tc_dense(y4, x4, w0, w1)
    out0 = o0[:, :, None]
    out1 = jnp.transpose(o1, (1, 2, 0))
    return out0, out1
